# Initial kernel scaffold; baseline (speedup 1.0000x reference)
import jax
import jax.numpy as jnp
from jax.experimental import pallas as pl


def kernel(xyz, params):
    raise NotImplementedError("write your pallas kernel here")


# speedup vs baseline: 3.2507x; 3.2507x over previous
import functools
import jax
import jax.numpy as jnp
from jax import lax
from jax.experimental import pallas as pl
from jax.experimental.pallas import tpu as pltpu
from jax.experimental.pallas import tpu_sc as plsc

_NC = 2
_NS = 16
_L = 16
_NW = _NC * _NS


def _fps_body(x_ref, o_ref, *, npoint):
    B, _, N = x_ref.shape
    x0 = x_ref[:, 0, :]
    x1 = x_ref[:, 1, :]
    x2 = x_ref[:, 2, :]
    iota_n = lax.broadcasted_iota(jnp.int32, (B, N), 1)
    iota_s = lax.broadcasted_iota(jnp.int32, (B, npoint), 1)

    def step(i, carry):
        mind, far, o0, o1, o2 = carry
        oh = (iota_n == far).astype(jnp.float32)
        c0 = jnp.sum(x0 * oh, axis=1, keepdims=True)
        c1 = jnp.sum(x1 * oh, axis=1, keepdims=True)
        c2 = jnp.sum(x2 * oh, axis=1, keepdims=True)
        d0 = (x0 - c0) ** 2
        d1 = (x1 - c1) ** 2
        d2 = (x2 - c2) ** 2
        dist = (d0 + d1) + d2
        mind = jnp.minimum(mind, dist)
        m = jnp.max(mind, axis=1, keepdims=True)
        far_new = jnp.min(jnp.where(mind == m, iota_n, N), axis=1, keepdims=True)
        ohs = (iota_s == i).astype(jnp.float32)
        o0 = o0 + c0 * ohs
        o1 = o1 + c1 * ohs
        o2 = o2 + c2 * ohs
        return (mind, far_new, o0, o1, o2)

    zero_s = jnp.zeros((B, npoint), jnp.float32)
    init = (
        jnp.full((B, N), 1e10, jnp.float32),
        jnp.zeros((B, 1), jnp.int32),
        zero_s, zero_s, zero_s,
    )
    mind, far, o0, o1, o2 = lax.fori_loop(0, npoint, step, init)
    o_ref[:, 0, :] = o0
    o_ref[:, 1, :] = o1
    o_ref[:, 2, :] = o2


def _fps_pallas(xyz_cn, npoint):
    B, _, N = xyz_cn.shape
    return pl.pallas_call(
        functools.partial(_fps_body, npoint=npoint),
        out_shape=jax.ShapeDtypeStruct((B, 3, npoint), jnp.float32),
    )(xyz_cn)


def _mask_body(x_ref, s_ref, o_ref, *, r2):
    x = x_ref[0]
    s = s_ref[0]
    x0 = x[0:1, :]
    x1 = x[1:2, :]
    x2 = x[2:3, :]
    s0 = s[:, 0:1]
    s1 = s[:, 1:2]
    s2 = s[:, 2:3]
    xx = (x0 * x0 + x1 * x1) + x2 * x2
    ss = (s0 * s0 + s1 * s1) + s2 * s2
    cross = jnp.dot(s, x, preferred_element_type=jnp.float32)
    sq = (ss + xx) - 2.0 * cross
    o_ref[...] = (sq <= r2).astype(jnp.float32)


def _mask_pallas(xyz_cn, new_xyz_t, radius):
    B, _, N = xyz_cn.shape
    S = new_xyz_t.shape[1]
    SBLK = min(S, 256)
    nsb = S // SBLK
    return pl.pallas_call(
        functools.partial(_mask_body, r2=radius ** 2),
        grid=(B, nsb),
        in_specs=[
            pl.BlockSpec((1, 3, N), lambda b, sb: (b, 0, 0)),
            pl.BlockSpec((1, SBLK, 3), lambda b, sb: (b, sb, 0)),
        ],
        out_specs=pl.BlockSpec((SBLK, N), lambda b, sb: (b * nsb + sb, 0)),
        out_shape=jax.ShapeDtypeStruct((B * S, N), jnp.float32),
    )(xyz_cn, new_xyz_t)


def _compact_pallas(mask, S, N, nsample):
    R = mask.shape[0]
    r_pw = R // _NW
    mesh = plsc.VectorSubcoreMesh(core_axis_name="c", subcore_axis_name="s")

    @functools.partial(
        pl.kernel, mesh=mesh,
        out_type=jax.ShapeDtypeStruct((R, nsample), jnp.int32),
        scratch_types=[
            pltpu.VMEM((N,), jnp.float32),
            pltpu.VMEM((64,), jnp.int32),
            pltpu.VMEM((r_pw, nsample), jnp.int32),
        ],
        compiler_params=pltpu.CompilerParams(use_tc_tiling_on_sc=False,
                                             needs_layout_passes=False),
    )
    def k(mask_hbm, idx_hbm, mrow_v, buf_v, out_v):
        wid = lax.axis_index("s") * _NC + lax.axis_index("c")
        base = wid * r_pw
        iota = lax.iota(jnp.int32, 16)
        zeros16 = jnp.zeros((16,), jnp.int32)

        def row_body(j, _):
            r = base + j
            pltpu.sync_copy(mask_hbm.at[r], mrow_v)

            def body(g, wp):
                mv = mrow_v[pl.ds(g * _L, _L)]
                msk = jnp.logical_and(mv > 0.0, wp < nsample)
                mi = msk.astype(jnp.int32)
                pos = jnp.cumsum(mi)
                tgt = jnp.where(msk, wp + pos - 1, 63)
                plsc.store_scatter(buf_v, [tgt], iota + g * _L)
                return wp + jnp.sum(mi)

            wp = lax.fori_loop(0, N // _L, body, jnp.int32(0))
            count = jnp.minimum(wp, nsample)
            first = plsc.load_gather(buf_v, [zeros16])
            boff = (r // S) * N
            for t in range(nsample // _L):
                sl = buf_v[pl.ds(t * _L, _L)]
                io = iota + t * _L
                out_v[j, pl.ds(t * _L, _L)] = jnp.where(io < count, sl, first) + boff
            return 0

        lax.fori_loop(0, r_pw, row_body, 0)
        pltpu.sync_copy(out_v, idx_hbm.at[pl.ds(base, r_pw)])

    return k(mask).reshape(R * nsample)


def _gather_pallas(table, flat_idx):
    M = flat_idx.shape[0]
    Cp = table.shape[1]
    m_pw = M // _NW
    CH = min(128, m_pw)
    nch = m_pw // CH
    mesh = plsc.VectorSubcoreMesh(core_axis_name="c", subcore_axis_name="s")

    @functools.partial(
        pl.kernel, mesh=mesh,
        out_type=jax.ShapeDtypeStruct((M, Cp), jnp.float32),
        scratch_types=[
            pltpu.VMEM((CH,), jnp.int32),
            pltpu.VMEM((CH, Cp), jnp.float32),
            pltpu.SemaphoreType.DMA,
        ],
        compiler_params=pltpu.CompilerParams(use_tc_tiling_on_sc=False,
                                             needs_layout_passes=False),
    )
    def k(table_hbm, idx_hbm, out_hbm, idx_v, rows_v, sem):
        wid = lax.axis_index("s") * _NC + lax.axis_index("c")
        base = wid * m_pw

        def chunk(kk, _):
            off = base + kk * CH
            pltpu.sync_copy(idx_hbm.at[pl.ds(off, CH)], idx_v)
            pltpu.async_copy(table_hbm.at[idx_v], rows_v, sem).wait()
            pltpu.sync_copy(rows_v, out_hbm.at[pl.ds(off, CH)])
            return 0

        lax.fori_loop(0, nch, chunk, 0)

    return k(table, flat_idx)


_BLK = 1024


def _mm_body(x_ref, c_ref, wt_ref, b_ref, y_ref, st_ref, *, mode):
    if mode == "first":
        x = x_ref[...] - c_ref[...]
    else:
        x = jnp.maximum(x_ref[...] * c_ref[0:1, :] + c_ref[4:5, :], 0.0)
    y = jnp.dot(x, wt_ref[...], preferred_element_type=jnp.float32) + b_ref[0:1, :]
    y_ref[...] = y
    st_ref[0, 0:1, :] = jnp.sum(y, axis=0, keepdims=True)
    st_ref[0, 1:2, :] = jnp.sum(y * y, axis=0, keepdims=True)


def _mm_pallas(x, c, wt, b, mode):
    P, Ci = x.shape
    Co = wt.shape[1]
    nblk = P // _BLK
    if mode == "first":
        c_spec = pl.BlockSpec((_BLK, Ci), lambda i: (i, 0))
    else:
        c_spec = pl.BlockSpec((8, Ci), lambda i: (0, 0))
    y, st = pl.pallas_call(
        functools.partial(_mm_body, mode=mode),
        grid=(nblk,),
        in_specs=[
            pl.BlockSpec((_BLK, Ci), lambda i: (i, 0)),
            c_spec,
            pl.BlockSpec((Ci, Co), lambda i: (0, 0)),
            pl.BlockSpec((8, Co), lambda i: (0, 0)),
        ],
        out_specs=[
            pl.BlockSpec((_BLK, Co), lambda i: (i, 0)),
            pl.BlockSpec((1, 2, Co), lambda i: (i, 0, 0)),
        ],
        out_shape=[
            jax.ShapeDtypeStruct((P, Co), jnp.float32),
            jax.ShapeDtypeStruct((nblk, 2, Co), jnp.float32),
        ],
    )(x, c, wt, b)
    return y, jnp.sum(st, axis=0)


def _pool_body(x_ref, c_ref, o_ref, *, ns, C):
    f = jnp.maximum(x_ref[...] * c_ref[0:1, :] + c_ref[4:5, :], 0.0)
    acc = f[:, 0:C]
    for k in range(1, ns):
        acc = jnp.maximum(acc, f[:, k * C:(k + 1) * C])
    o_ref[...] = acc


def _pool_pallas(x, c, ns):
    G, W = x.shape
    C = W // ns
    BLK2 = min(G, 256)
    return pl.pallas_call(
        functools.partial(_pool_body, ns=ns, C=C),
        grid=(G // BLK2,),
        in_specs=[
            pl.BlockSpec((BLK2, W), lambda i: (i, 0)),
            pl.BlockSpec((8, W), lambda i: (0, 0)),
        ],
        out_specs=pl.BlockSpec((BLK2, C), lambda i: (i, 0)),
        out_shape=jax.ShapeDtypeStruct((G, C), jnp.float32),
    )(x, c)


def _bn_coeffs(st, P, gamma, beta, reps=1):
    mean = st[0] / P
    var = st[1] / P - mean * mean
    scale = gamma / jnp.sqrt(var + 1e-5)
    shift = beta - mean * scale
    scale = jnp.tile(scale, reps)
    shift = jnp.tile(shift, reps)
    return jnp.concatenate([jnp.broadcast_to(scale[None, :], (4, scale.shape[0])),
                            jnp.broadcast_to(shift[None, :], (4, shift.shape[0]))], axis=0)


def _row8(v):
    return jnp.broadcast_to(v[None, :], (8, v.shape[0]))


def mlp_chain(g, ce, layer_params, ns):
    P = g.shape[0]
    Cp = g.shape[1]
    (W1, b1, g1, be1), (W2, b2, g2, be2), (W3, b3, g3, be3) = layer_params
    w1t = jnp.zeros((Cp, W1.shape[0]), jnp.float32).at[: W1.shape[1], :].set(W1.T)
    y1, st1 = _mm_pallas(g, ce, w1t, _row8(b1), "first")
    c1 = _bn_coeffs(st1, P, g1, be1)
    y2, st2 = _mm_pallas(y1, c1, W2.T, _row8(b2), "mid")
    c2 = _bn_coeffs(st2, P, g2, be2)
    y3, st3 = _mm_pallas(y2, c2, W3.T, _row8(b3), "mid")
    C3 = y3.shape[1]
    c3 = _bn_coeffs(st3, P, g3, be3, reps=ns)
    return _pool_pallas(y3.reshape(P // ns, ns * C3), c3, ns)


def _pad_cols(x, cp):
    pad = cp - x.shape[-1]
    if pad == 0:
        return x
    return jnp.concatenate([x, jnp.zeros(x.shape[:-1] + (pad,), x.dtype)], axis=-1)


def _set_abstraction(xyz, points, npoint, radius, nsample, layer_params):
    B, _, N = xyz.shape
    D = points.shape[1]
    S = npoint
    new_xyz_cn = _fps_pallas(xyz, npoint)
    new_xyz_t = jnp.transpose(new_xyz_cn, (0, 2, 1))

    xyz_tt = jnp.transpose(xyz, (0, 2, 1))
    sqd = (jnp.sum(new_xyz_t ** 2, axis=-1)[..., None]
           + jnp.sum(xyz_tt ** 2, axis=-1)[..., None, :]
           - 2.0 * jnp.matmul(new_xyz_t, jnp.transpose(xyz_tt, (0, 2, 1))))
    mask = (sqd <= radius ** 2).astype(jnp.float32).reshape(B * S, N)
    key = jnp.where(mask > 0, jnp.broadcast_to(jnp.arange(N, dtype=jnp.int32), mask.shape), N)
    key = jnp.sort(key, axis=-1)[:, :nsample]
    first = key[:, 0:1]
    key = jnp.where(key == N, first, key)
    boff = (jnp.arange(B * S, dtype=jnp.int32) // S * N)[:, None]
    flat_idx = (key + boff).reshape(B * S * nsample)

    Craw = 3 + D
    Cp = ((Craw + _L - 1) // _L) * _L
    xyz_t = jnp.transpose(xyz, (0, 2, 1))
    pts_t = jnp.transpose(points, (0, 2, 1))
    table = _pad_cols(jnp.concatenate([xyz_t, pts_t], axis=-1), Cp)
    table = table.reshape(B * N, Cp)

    g = jnp.take(table, flat_idx, axis=0)

    ce = jnp.concatenate(
        [new_xyz_t.reshape(B * S, 3), jnp.zeros((B * S, Cp - 3), jnp.float32)],
        axis=-1)
    ce_exp = jnp.repeat(ce, nsample, axis=0)
    pooled = mlp_chain(g.reshape(B * S * nsample, Cp), ce_exp, layer_params,
                       nsample)
    new_points_out = jnp.transpose(pooled.reshape(B, S, -1), (0, 2, 1))
    return new_xyz_cn, new_points_out


def kernel(xyz, params):
    l0_points = xyz
    l0_xyz = xyz[:, :3, :]
    l1_xyz, l1_points = _set_abstraction(l0_xyz, l0_points, 1024, 0.1, 32, params['sa1'])
    l2_xyz, l2_points = _set_abstraction(l1_xyz, l1_points, 256, 0.2, 32, params['sa2'])
    l3_xyz, l3_points = _set_abstraction(l2_xyz, l2_points, 64, 0.4, 32, params['sa3'])
    l4_xyz, l4_points = _set_abstraction(l3_xyz, l3_points, 16, 0.8, 32, params['sa4'])
    return (l4_xyz, l4_points)
